# baseline (speedup 1.0000x reference)
import jax
import jax.numpy as jnp
from jax.experimental import pallas as pl
from jax.experimental.pallas import tpu as pltpu

_G = 52
_GG = _G * _G
_C = 25
_STRIDE = 8.0
_NB = 4


def _body(anchor_ref, x_ref, out_ref):
    a = pl.program_id(1)
    p = jax.lax.broadcasted_iota(
        jnp.int32, (1, _GG), 1).astype(jnp.float32)
    gy = jnp.floor((p + 0.5) * (1.0 / _G))
    gx = p - _G * gy
    grid01 = jnp.concatenate([gx, gy], axis=0)
    sc23 = jnp.concatenate(
        [jnp.full((1, 1), anchor_ref[a, 0], jnp.float32),
         jnp.full((1, 1), anchor_ref[a, 1], jnp.float32)], axis=0)

    for n in range(_NB):
        xa = x_ref[n].reshape(_C, _GG)
        xy = (jax.nn.sigmoid(xa[0:2]) + grid01) * _STRIDE
        wh = jnp.exp(xa[2:4]) * sc23
        rest = jax.nn.sigmoid(xa[4:_C])
        y = jnp.concatenate([xy, wh, rest], axis=0)
        out_ref[n] = y.T


def kernel(x, anchor_wh):
    B = x.shape[0]
    out = pl.pallas_call(
        _body,
        grid=(B // _NB, 3),
        in_specs=[
            pl.BlockSpec(memory_space=pltpu.SMEM),
            pl.BlockSpec((_NB, _C, _G, _G), lambda b, a: (b, a, 0, 0)),
        ],
        out_specs=pl.BlockSpec((_NB, _GG, _C), lambda b, a: (b, a, 0)),
        out_shape=jax.ShapeDtypeStruct((B, 3 * _GG, _C), jnp.float32),
        compiler_params=pltpu.CompilerParams(
            dimension_semantics=("parallel", "arbitrary"),
        ),
    )(anchor_wh, x)
    return out
